# baseline (speedup 1.0000x reference)
import functools

import jax
import jax.numpy as jnp
from jax import lax
from jax.experimental import pallas as pl
from jax.experimental.pallas import tpu as pltpu
from jax.experimental.pallas import tpu_sc as plsc

N_EDGES = 100000
KMAX = 8
N_SPH = 8
N_RAGGED = 400000
EMB = 32
EMB_INT = 16
UNITS_OUT = 32

NW = 32
WE = 250
WSLOTS = WE * KMAX
NWIN = N_EDGES // WE
WMAX = (NWIN + NW - 1) // NW
SEG = 1024
WT_PAD = 2000
NB_PAD = 416


def _sc_build_m2(idr_hbm, idk_hbm, m_hbm, wb_hbm, m2_hbm,
                 wt_v, idr_v, idk_v, mstage_v, m2buf_v, wb_v,
                 sem_i, sem_w):
    wid = lax.axis_index("s") * 2 + lax.axis_index("c")
    lanes = lax.iota(jnp.int32, 16)
    zeros16 = jnp.zeros((16,), jnp.int32)

    pltpu.sync_copy(wb_hbm, wb_v)

    def pickb(j):
        jv = jnp.full((16,), 0, jnp.int32) + j
        return jnp.max(plsc.load_gather(wb_v, [jv]))

    def window_body(w, carry):
        gw = w * NW + wid
        we0 = gw * WE
        we1 = we0 + WE
        live = gw < NWIN
        rs = pickb(jnp.minimum(gw, NWIN - 1))
        re = pickb(jnp.minimum(gw + 1, NWIN))
        rs8 = (rs // 8) * 8
        nseg = jnp.where(live,
                         jnp.maximum((re - rs8 + (SEG - 1)) // SEG, 1), 0)

        def seg_body(si, carry2):
            rb_u = rs8 + si * SEG
            rb = jnp.minimum(rb_u, N_RAGGED - SEG)
            cp_r = pltpu.async_copy(idr_hbm.at[pl.ds(rb, SEG)], idr_v, sem_i)
            cp_k = pltpu.async_copy(idk_hbm.at[pl.ds(rb, SEG)], idk_v, sem_i)
            cp_m = pltpu.async_copy(m_hbm.at[pl.ds(rb, SEG)], mstage_v, sem_i)

            def ztbody(i, c3):
                wt_v[pl.ds(i * 16, 16)] = zeros16
                return c3
            lax.fori_loop(0, WT_PAD // 16, ztbody, 0)
            cp_r.wait()
            cp_k.wait()
            cp_m.wait()

            def step(j, c3):
                ids = idr_v[pl.ds(j * 16, 16)]
                ks = idk_v[pl.ds(j * 16, 16)]
                rloc = j * 16 + lanes
                valid = ((ids >= we0) & (ids < we1)
                         & (rb + rloc >= rb_u))
                key = (ids - we0) * 8 + ks
                keyc = jnp.where(valid, key, WSLOTS)
                haslater = lanes < 0
                for sh in range(1, 16):
                    idx = (lanes + sh) % 16
                    rk = keyc.at[idx].get(mode="promise_in_bounds")
                    haslater = haslater | ((lanes < 16 - sh) & (rk == keyc))
                ok = valid & (~haslater)
                plsc.store_scatter(wt_v, [keyc], rloc + 1, mask=ok)
                return c3
            lax.fori_loop(0, SEG // 16, step, 0)

            @pl.when((w > 0) & (si == 0))
            def _wait_prev():
                pltpu.make_async_copy(
                    m2buf_v.at[pl.ds(0, WE)],
                    m2_hbm.at[pl.ds((gw - NW) * WE, WE)], sem_w).wait()

            first = si == 0

            def slotvec(sv, c3):
                slot = sv * 16 + lanes
                slot_ok = slot < WSLOTS
                w16 = wt_v[pl.ds(sv * 16, 16)]
                mw = w16 > 0
                rowloc = jnp.maximum(w16 - 1, 0)
                wmask = slot_ok & (mw | first)
                rowv = slot >> 3
                colbase = (slot & 7) * EMB
                for e in range(EMB):
                    col = jnp.full((16,), 0, jnp.int32) + e
                    vals = plsc.load_gather(mstage_v, [rowloc, col], mask=mw)
                    vals = jnp.where(mw, vals, 0.0)
                    plsc.store_scatter(m2buf_v, [rowv, colbase + e], vals,
                                       mask=wmask)
                return c3
            lax.fori_loop(0, WT_PAD // 16, slotvec, 0)
            return carry2
        lax.fori_loop(0, nseg, seg_body, 0)

        @pl.when(live)
        def _issue_write():
            pltpu.async_copy(m2buf_v.at[pl.ds(0, WE)],
                             m2_hbm.at[pl.ds(we0, WE)], sem_w)
        return carry
    lax.fori_loop(0, WMAX, window_body, 0)

    last_gw = wid + NW * ((NWIN - 1 - wid) // NW)
    pltpu.make_async_copy(m2buf_v.at[pl.ds(0, WE)],
                          m2_hbm.at[pl.ds(last_gw * WE, WE)], sem_w).wait()


def _build_m2(id_reduce, id_ragged_idx, m):
    wb = jnp.searchsorted(
        id_reduce, jnp.arange(NWIN + 1, dtype=jnp.int32) * WE,
        side="left").astype(jnp.int32)
    wb = jnp.concatenate(
        [wb, jnp.zeros((NB_PAD - NWIN - 1,), jnp.int32)])

    mesh = plsc.VectorSubcoreMesh(core_axis_name="c", subcore_axis_name="s")
    sc_call = functools.partial(
        pl.kernel, mesh=mesh,
        compiler_params=pltpu.CompilerParams(
            needs_layout_passes=False, use_tc_tiling_on_sc=False),
        out_type=jax.ShapeDtypeStruct((N_EDGES, KMAX * EMB), jnp.float32),
        scratch_types=[
            pltpu.VMEM((WT_PAD,), jnp.int32),
            pltpu.VMEM((SEG,), jnp.int32),
            pltpu.VMEM((SEG,), jnp.int32),
            pltpu.VMEM((SEG, EMB), jnp.float32),
            pltpu.VMEM((WE + 6, KMAX * EMB), jnp.float32),
            pltpu.VMEM((NB_PAD,), jnp.int32),
            pltpu.SemaphoreType.DMA,
            pltpu.SemaphoreType.DMA,
        ])(_sc_build_m2)
    return sc_call(id_reduce, id_ragged_idx, m, wb)


BE = 1000


def _tc_body(sph_ref, rbf_ref, m2_ref, w2t_ref, out_ref):
    sphT = sph_ref[...].T
    rbfT = rbf_ref[...].T
    m2T = m2_ref[...].T
    g = []
    for s in range(N_SPH):
        acc = sphT[s * 8:s * 8 + 1, :] * m2T[0:EMB, :]
        for k in range(1, KMAX):
            acc = acc + sphT[s * 8 + k:s * 8 + k + 1, :] * \
                m2T[k * EMB:(k + 1) * EMB, :]
        g.append(acc)
    d = []
    for i in range(EMB_INT):
        acc = rbfT[i * 8:i * 8 + 1, :] * g[0]
        for s in range(1, N_SPH):
            acc = acc + rbfT[i * 8 + s:i * 8 + s + 1, :] * g[s]
        d.append(acc)
    dT = jnp.concatenate(d, axis=0)
    outT = jnp.dot(w2t_ref[...], dT,
                   preferred_element_type=jnp.float32)
    out_ref[...] = outT.T


def kernel(rbf_W1, sph, m, id_reduce, id_ragged_idx, weight):
    m2r = _build_m2(id_reduce, id_ragged_idx, m)
    sph2 = sph.reshape(N_EDGES, N_SPH * KMAX)
    rbf2 = rbf_W1.reshape(N_EDGES, EMB_INT * N_SPH)
    w2t = jnp.transpose(weight, (1, 0, 2)).reshape(
        EMB_INT * EMB, UNITS_OUT).T

    grid = N_EDGES // BE
    out = pl.pallas_call(
        _tc_body,
        grid=(grid,),
        in_specs=[
            pl.BlockSpec((BE, N_SPH * KMAX), lambda i: (i, 0)),
            pl.BlockSpec((BE, EMB_INT * N_SPH), lambda i: (i, 0)),
            pl.BlockSpec((BE, KMAX * EMB), lambda i: (i, 0)),
            pl.BlockSpec((UNITS_OUT, EMB_INT * EMB), lambda i: (0, 0)),
        ],
        out_specs=pl.BlockSpec((BE, UNITS_OUT), lambda i: (i, 0)),
        out_shape=jax.ShapeDtypeStruct((N_EDGES, UNITS_OUT), jnp.float32),
    )(sph2, rbf2, m2r, w2t)
    return out
